# Initial kernel scaffold; baseline (speedup 1.0000x reference)
import jax
import jax.numpy as jnp
from jax.experimental import pallas as pl


def kernel(x, table, Wih0f, Whh0f, bih0f, bhh0f, Wih0b, Whh0b, bih0b, bhh0b, Wih1f, Whh1f, bih1f, bhh1f, Wih1b, Whh1b, bih1b, bhh1b, Wfc, bfc):
    raise NotImplementedError("write your pallas kernel here")


# speedup vs baseline: 5.4827x; 5.4827x over previous
import functools

import jax
import jax.numpy as jnp
from jax import lax
from jax.experimental import pallas as pl
from jax.experimental.pallas import tpu as pltpu
from jax.experimental.pallas import tpu_sc as plsc

VOCAB = 100000
DIM = 256
H = 256
CLASSES = 10
B = 1024
T = 50


def _sc_gather(table, idx):
    info = plsc.get_sparse_core_info()
    nc, ns = info.num_cores, info.num_subcores
    nw = nc * ns
    n = idx.shape[0]
    d = table.shape[1]
    per_w = n // nw
    ch = 200
    n_ch = per_w // ch
    mesh = plsc.VectorSubcoreMesh(core_axis_name="c", subcore_axis_name="s")

    @functools.partial(
        pl.kernel,
        mesh=mesh,
        out_type=jax.ShapeDtypeStruct((n, d), jnp.float32),
        scratch_types=[
            pltpu.VMEM((ch,), jnp.int32),
            pltpu.VMEM((ch, d), jnp.float32),
            pltpu.SemaphoreType.DMA,
        ],
    )
    def k(table_hbm, idx_hbm, out_hbm, idx_v, rows_v, sem):
        wid = lax.axis_index("s") * nc + lax.axis_index("c")
        for c_i in range(n_ch):
            base = wid * per_w + c_i * ch
            pltpu.sync_copy(idx_hbm.at[pl.ds(base, ch)], idx_v)
            pltpu.async_copy(table_hbm.at[idx_v], rows_v, sem).wait()
            pltpu.sync_copy(rows_v, out_hbm.at[pl.ds(base, ch)])

    return k(table, idx)


def _lstm_gates(gates, c):
    ii = jax.nn.sigmoid(gates[:, :H])
    ff = jax.nn.sigmoid(gates[:, H:2 * H])
    gg = jnp.tanh(gates[:, 2 * H:3 * H])
    oo = jax.nn.sigmoid(gates[:, 3 * H:])
    c2 = ff * c + ii * gg
    h2 = oo * jnp.tanh(c2)
    return h2, c2


def _l0_kernel(emb_ref, wih_ref, whh_ref, b_ref, out_ref, h_ref, c_ref):
    t = pl.program_id(1)

    @pl.when(t == 0)
    def _():
        h_ref[...] = jnp.zeros_like(h_ref)
        c_ref[...] = jnp.zeros_like(c_ref)

    gates = (
        jnp.dot(emb_ref[0], wih_ref[0], preferred_element_type=jnp.float32)
        + jnp.dot(h_ref[...], whh_ref[0], preferred_element_type=jnp.float32)
        + b_ref[0]
    )
    h2, c2 = _lstm_gates(gates, c_ref[...])
    h_ref[...] = h2
    c_ref[...] = c2
    out_ref[0, 0] = h2


def _layer0(emb, wih2, whh2, b2):
    def emb_map(di, t):
        return (jnp.where(di == 0, t, T - 1 - t), 0, 0)

    def out_map(di, t):
        return (di, jnp.where(di == 0, t, T - 1 - t), 0, 0)

    return pl.pallas_call(
        _l0_kernel,
        grid=(2, T),
        in_specs=[
            pl.BlockSpec((1, B, DIM), emb_map),
            pl.BlockSpec((1, DIM, 4 * H), lambda di, t: (di, 0, 0)),
            pl.BlockSpec((1, H, 4 * H), lambda di, t: (di, 0, 0)),
            pl.BlockSpec((1, 1, 4 * H), lambda di, t: (di, 0, 0)),
        ],
        out_specs=pl.BlockSpec((1, 1, B, H), out_map),
        out_shape=jax.ShapeDtypeStruct((2, T, B, H), jnp.float32),
        scratch_shapes=[
            pltpu.VMEM((B, H), jnp.float32),
            pltpu.VMEM((B, H), jnp.float32),
        ],
    )(emb, wih2, whh2, b2)


def _l1_kernel(hf_ref, hb_ref, wa_ref, wb_ref, whh_ref, b_ref, out_ref,
               h_ref, c_ref):
    t = pl.program_id(0)

    @pl.when(t == 0)
    def _():
        h_ref[...] = jnp.zeros_like(h_ref)
        c_ref[...] = jnp.zeros_like(c_ref)

    gates = (
        jnp.dot(hf_ref[0], wa_ref[...], preferred_element_type=jnp.float32)
        + jnp.dot(hb_ref[0], wb_ref[...], preferred_element_type=jnp.float32)
        + jnp.dot(h_ref[...], whh_ref[...], preferred_element_type=jnp.float32)
        + b_ref[...]
    )
    h2, c2 = _lstm_gates(gates, c_ref[...])
    h_ref[...] = h2
    c_ref[...] = c2

    @pl.when(t == T - 1)
    def _():
        out_ref[...] = h2


def _layer1_fwd(hf0, hb0, wa, wb, whh, b):
    return pl.pallas_call(
        _l1_kernel,
        grid=(T,),
        in_specs=[
            pl.BlockSpec((1, B, H), lambda t: (t, 0, 0)),
            pl.BlockSpec((1, B, H), lambda t: (t, 0, 0)),
            pl.BlockSpec((H, 4 * H), lambda t: (0, 0)),
            pl.BlockSpec((H, 4 * H), lambda t: (0, 0)),
            pl.BlockSpec((H, 4 * H), lambda t: (0, 0)),
            pl.BlockSpec((1, 4 * H), lambda t: (0, 0)),
        ],
        out_specs=pl.BlockSpec((B, H), lambda t: (0, 0)),
        out_shape=jax.ShapeDtypeStruct((B, H), jnp.float32),
        scratch_shapes=[
            pltpu.VMEM((B, H), jnp.float32),
            pltpu.VMEM((B, H), jnp.float32),
        ],
    )(hf0, hb0, wa, wb, whh, b)


def _final_kernel(hfl_ref, hbl_ref, hf1_ref, wa_ref, wb_ref, b_ref,
                  wfa_ref, wfb_ref, bfc_ref, out_ref):
    gates = (
        jnp.dot(hfl_ref[...], wa_ref[...], preferred_element_type=jnp.float32)
        + jnp.dot(hbl_ref[...], wb_ref[...], preferred_element_type=jnp.float32)
        + b_ref[...]
    )
    h2, _ = _lstm_gates(gates, jnp.zeros_like(hfl_ref[...]))
    out_ref[...] = (
        jnp.dot(hf1_ref[...], wfa_ref[...], preferred_element_type=jnp.float32)
        + jnp.dot(h2, wfb_ref[...], preferred_element_type=jnp.float32)
        + bfc_ref[...]
    )


def _final(hf0_last, hb0_last, hf1, wa, wb, b, wfa, wfb, bfc_row):
    npad = wfa.shape[1]
    return pl.pallas_call(
        _final_kernel,
        out_shape=jax.ShapeDtypeStruct((B, npad), jnp.float32),
    )(hf0_last, hb0_last, hf1, wa, wb, b, wfa, wfb, bfc_row)


def kernel(x, table, Wih0f, Whh0f, bih0f, bhh0f, Wih0b, Whh0b, bih0b, bhh0b,
           Wih1f, Whh1f, bih1f, bhh1f, Wih1b, Whh1b, bih1b, bhh1b, Wfc, bfc):
    idx = x.astype(jnp.int32).T.reshape(-1)
    emb = _sc_gather(table, idx).reshape(T, B, DIM)

    wih2 = jnp.stack([Wih0f.T, Wih0b.T])
    whh2 = jnp.stack([Whh0f.T, Whh0b.T])
    b2 = jnp.stack([(bih0f + bhh0f).reshape(1, -1),
                    (bih0b + bhh0b).reshape(1, -1)])
    h0 = _layer0(emb, wih2, whh2, b2)
    hf0, hb0 = h0[0], h0[1]

    w1f = Wih1f.T
    hf1 = _layer1_fwd(hf0, hb0, w1f[:H], w1f[H:], Whh1f.T,
                      (bih1f + bhh1f).reshape(1, -1))

    w1b = Wih1b.T
    npad = 128
    wfc_t = jnp.zeros((2 * H, npad), jnp.float32).at[:, :CLASSES].set(Wfc.T)
    bfc_row = jnp.zeros((1, npad), jnp.float32).at[:, :CLASSES].set(bfc)
    logits = _final(hf0[T - 1], hb0[T - 1], hf1,
                    w1b[:H], w1b[H:], (bih1b + bhh1b).reshape(1, -1),
                    wfc_t[:H], wfc_t[H:], bfc_row)
    return logits[:, :CLASSES]


# speedup vs baseline: 5.9316x; 1.0819x over previous
import functools

import jax
import jax.numpy as jnp
from jax import lax
from jax.experimental import pallas as pl
from jax.experimental.pallas import tpu as pltpu
from jax.experimental.pallas import tpu_sc as plsc

VOCAB = 100000
DIM = 256
H = 256
CLASSES = 10
B = 1024
T = 50


def _sc_gather(table, idx):
    info = plsc.get_sparse_core_info()
    nc, ns = info.num_cores, info.num_subcores
    nw = nc * ns
    n = idx.shape[0]
    d = table.shape[1]
    per_w = n // nw
    ch = 200
    n_ch = per_w // ch
    mesh = plsc.VectorSubcoreMesh(core_axis_name="c", subcore_axis_name="s")

    @functools.partial(
        pl.kernel,
        mesh=mesh,
        out_type=jax.ShapeDtypeStruct((n, d), jnp.float32),
        scratch_types=[
            pltpu.VMEM((ch,), jnp.int32),
            pltpu.VMEM((ch, d), jnp.float32),
            pltpu.SemaphoreType.DMA,
        ],
    )
    def k(table_hbm, idx_hbm, out_hbm, idx_v, rows_v, sem):
        wid = lax.axis_index("s") * nc + lax.axis_index("c")
        for c_i in range(n_ch):
            base = wid * per_w + c_i * ch
            pltpu.sync_copy(idx_hbm.at[pl.ds(base, ch)], idx_v)
            pltpu.async_copy(table_hbm.at[idx_v], rows_v, sem).wait()
            pltpu.sync_copy(rows_v, out_hbm.at[pl.ds(base, ch)])

    return k(table, idx)


def _lstm_gates(gates, c):
    ii = jax.nn.sigmoid(gates[:, :H])
    ff = jax.nn.sigmoid(gates[:, H:2 * H])
    gg = jnp.tanh(gates[:, 2 * H:3 * H])
    oo = jax.nn.sigmoid(gates[:, 3 * H:])
    c2 = ff * c + ii * gg
    h2 = oo * jnp.tanh(c2)
    return h2, c2


def _l0_kernel(emb_ref, wih_ref, whh_ref, b_ref, out_ref, h_ref, c_ref):
    t = pl.program_id(1)

    @pl.when(t == 0)
    def _():
        h_ref[...] = jnp.zeros_like(h_ref)
        c_ref[...] = jnp.zeros_like(c_ref)

    gates = (
        jnp.dot(emb_ref[0].astype(jnp.bfloat16), wih_ref[0],
                preferred_element_type=jnp.float32)
        + jnp.dot(h_ref[...], whh_ref[0], preferred_element_type=jnp.float32)
        + b_ref[0]
    )
    h2, c2 = _lstm_gates(gates, c_ref[...])
    h2b = h2.astype(jnp.bfloat16)
    h_ref[...] = h2b
    c_ref[...] = c2
    out_ref[0, 0] = h2b


def _layer0(emb, wih2, whh2, b2):
    def emb_map(di, t):
        return (jnp.where(di == 0, t, T - 1 - t), 0, 0)

    def out_map(di, t):
        return (di, jnp.where(di == 0, t, T - 1 - t), 0, 0)

    return pl.pallas_call(
        _l0_kernel,
        grid=(2, T),
        in_specs=[
            pl.BlockSpec((1, B, DIM), emb_map),
            pl.BlockSpec((1, DIM, 4 * H), lambda di, t: (di, 0, 0)),
            pl.BlockSpec((1, H, 4 * H), lambda di, t: (di, 0, 0)),
            pl.BlockSpec((1, 1, 4 * H), lambda di, t: (di, 0, 0)),
        ],
        out_specs=pl.BlockSpec((1, 1, B, H), out_map),
        out_shape=jax.ShapeDtypeStruct((2, T, B, H), jnp.bfloat16),
        scratch_shapes=[
            pltpu.VMEM((B, H), jnp.bfloat16),
            pltpu.VMEM((B, H), jnp.float32),
        ],
    )(emb, wih2, whh2, b2)


def _l1_kernel(hf_ref, hb_ref, wa_ref, wb_ref, whh_ref, b_ref, out_ref,
               h_ref, c_ref):
    t = pl.program_id(0)

    @pl.when(t == 0)
    def _():
        h_ref[...] = jnp.zeros_like(h_ref)
        c_ref[...] = jnp.zeros_like(c_ref)

    gates = (
        jnp.dot(hf_ref[0], wa_ref[...], preferred_element_type=jnp.float32)
        + jnp.dot(hb_ref[0], wb_ref[...], preferred_element_type=jnp.float32)
        + jnp.dot(h_ref[...], whh_ref[...], preferred_element_type=jnp.float32)
        + b_ref[...]
    )
    h2, c2 = _lstm_gates(gates, c_ref[...])
    h2b = h2.astype(jnp.bfloat16)
    h_ref[...] = h2b
    c_ref[...] = c2

    @pl.when(t == T - 1)
    def _():
        out_ref[...] = h2b


def _layer1_fwd(hf0, hb0, wa, wb, whh, b):
    return pl.pallas_call(
        _l1_kernel,
        grid=(T,),
        in_specs=[
            pl.BlockSpec((1, B, H), lambda t: (t, 0, 0)),
            pl.BlockSpec((1, B, H), lambda t: (t, 0, 0)),
            pl.BlockSpec((H, 4 * H), lambda t: (0, 0)),
            pl.BlockSpec((H, 4 * H), lambda t: (0, 0)),
            pl.BlockSpec((H, 4 * H), lambda t: (0, 0)),
            pl.BlockSpec((1, 4 * H), lambda t: (0, 0)),
        ],
        out_specs=pl.BlockSpec((B, H), lambda t: (0, 0)),
        out_shape=jax.ShapeDtypeStruct((B, H), jnp.bfloat16),
        scratch_shapes=[
            pltpu.VMEM((B, H), jnp.bfloat16),
            pltpu.VMEM((B, H), jnp.float32),
        ],
    )(hf0, hb0, wa, wb, whh, b)


def _final_kernel(hfl_ref, hbl_ref, hf1_ref, wa_ref, wb_ref, b_ref,
                  wfa_ref, wfb_ref, bfc_ref, out_ref):
    gates = (
        jnp.dot(hfl_ref[...], wa_ref[...], preferred_element_type=jnp.float32)
        + jnp.dot(hbl_ref[...], wb_ref[...], preferred_element_type=jnp.float32)
        + b_ref[...]
    )
    h2, _ = _lstm_gates(gates, jnp.zeros((B, H), jnp.float32))
    h2 = h2.astype(jnp.bfloat16)
    out_ref[...] = (
        jnp.dot(hf1_ref[...], wfa_ref[...], preferred_element_type=jnp.float32)
        + jnp.dot(h2, wfb_ref[...], preferred_element_type=jnp.float32)
        + bfc_ref[...]
    )


def _final(hf0_last, hb0_last, hf1, wa, wb, b, wfa, wfb, bfc_row):
    npad = wfa.shape[1]
    return pl.pallas_call(
        _final_kernel,
        out_shape=jax.ShapeDtypeStruct((B, npad), jnp.float32),
    )(hf0_last, hb0_last, hf1, wa, wb, b, wfa, wfb, bfc_row)


def kernel(x, table, Wih0f, Whh0f, bih0f, bhh0f, Wih0b, Whh0b, bih0b, bhh0b,
           Wih1f, Whh1f, bih1f, bhh1f, Wih1b, Whh1b, bih1b, bhh1b, Wfc, bfc):
    idx = x.astype(jnp.int32).T.reshape(-1)
    emb = _sc_gather(table, idx).reshape(T, B, DIM)

    bf = jnp.bfloat16
    wih2 = jnp.stack([Wih0f.T, Wih0b.T]).astype(bf)
    whh2 = jnp.stack([Whh0f.T, Whh0b.T]).astype(bf)
    b2 = jnp.stack([(bih0f + bhh0f).reshape(1, -1),
                    (bih0b + bhh0b).reshape(1, -1)])
    h0 = _layer0(emb, wih2, whh2, b2)
    hf0, hb0 = h0[0], h0[1]

    w1f = Wih1f.T.astype(bf)
    hf1 = _layer1_fwd(hf0, hb0, w1f[:H], w1f[H:], Whh1f.T.astype(bf),
                      (bih1f + bhh1f).reshape(1, -1))

    w1b = Wih1b.T.astype(bf)
    npad = 128
    wfc_t = jnp.zeros((2 * H, npad), jnp.float32).at[:, :CLASSES].set(Wfc.T)
    wfc_t = wfc_t.astype(bf)
    bfc_row = jnp.zeros((1, npad), jnp.float32).at[:, :CLASSES].set(bfc)
    logits = _final(hf0[T - 1], hb0[T - 1], hf1,
                    w1b[:H], w1b[H:], (bih1b + bhh1b).reshape(1, -1),
                    wfc_t[:H], wfc_t[H:], bfc_row)
    return logits[:, :CLASSES]


# speedup vs baseline: 6.6580x; 1.1225x over previous
import functools

import jax
import jax.numpy as jnp
from jax import lax
from jax.experimental import pallas as pl
from jax.experimental.pallas import tpu as pltpu
from jax.experimental.pallas import tpu_sc as plsc

VOCAB = 100000
DIM = 256
H = 256
CLASSES = 10
B = 1024
T = 50


def _sc_gather(table, idx):
    info = plsc.get_sparse_core_info()
    nc, ns = info.num_cores, info.num_subcores
    nw = nc * ns
    n = idx.shape[0]
    d = table.shape[1]
    per_w = n // nw
    ch = 200
    n_ch = per_w // ch
    mesh = plsc.VectorSubcoreMesh(core_axis_name="c", subcore_axis_name="s")

    @functools.partial(
        pl.kernel,
        mesh=mesh,
        out_type=jax.ShapeDtypeStruct((n, d), jnp.float32),
        scratch_types=[
            pltpu.VMEM((ch,), jnp.int32),
            pltpu.VMEM((ch, d), jnp.float32),
            pltpu.SemaphoreType.DMA,
        ],
    )
    def k(table_hbm, idx_hbm, out_hbm, idx_v, rows_v, sem):
        wid = lax.axis_index("s") * nc + lax.axis_index("c")
        for c_i in range(n_ch):
            base = wid * per_w + c_i * ch
            pltpu.sync_copy(idx_hbm.at[pl.ds(base, ch)], idx_v)
            pltpu.async_copy(table_hbm.at[idx_v], rows_v, sem).wait()
            pltpu.sync_copy(rows_v, out_hbm.at[pl.ds(base, ch)])

    return k(table, idx)


def _lstm_gates(gates, c):
    ii = 0.5 * jnp.tanh(gates[:, :H]) + 0.5
    ff = 0.5 * jnp.tanh(gates[:, H:2 * H]) + 0.5
    gg = jnp.tanh(gates[:, 2 * H:3 * H])
    oo = 0.5 * jnp.tanh(gates[:, 3 * H:]) + 0.5
    c2 = ff * c + ii * gg
    h2 = oo * jnp.tanh(c2)
    return h2, c2


def _halve_ifo(wt):
    s = jnp.concatenate([jnp.full((2 * H,), 0.5, jnp.float32),
                         jnp.ones((H,), jnp.float32),
                         jnp.full((H,), 0.5, jnp.float32)])
    return wt * s


def _halve_ifo_bias(b_row):
    s = jnp.concatenate([jnp.full((2 * H,), 0.5, jnp.float32),
                         jnp.ones((H,), jnp.float32),
                         jnp.full((H,), 0.5, jnp.float32)])
    return b_row * s


def _l0_kernel(emb_ref, wih_ref, whh_ref, b_ref, out_ref, h_ref, c_ref):
    t = pl.program_id(1)

    @pl.when(t == 0)
    def _():
        h_ref[...] = jnp.zeros_like(h_ref)
        c_ref[...] = jnp.zeros_like(c_ref)

    gates = (
        jnp.dot(emb_ref[0].astype(jnp.bfloat16), wih_ref[0],
                preferred_element_type=jnp.float32)
        + jnp.dot(h_ref[...], whh_ref[0], preferred_element_type=jnp.float32)
        + b_ref[0]
    )
    h2, c2 = _lstm_gates(gates, c_ref[...])
    h2b = h2.astype(jnp.bfloat16)
    h_ref[...] = h2b
    c_ref[...] = c2
    out_ref[0, 0] = h2b


def _layer0(emb, wih2, whh2, b2):
    def emb_map(di, t):
        return (jnp.where(di == 0, t, T - 1 - t), 0, 0)

    def out_map(di, t):
        return (di, jnp.where(di == 0, t, T - 1 - t), 0, 0)

    return pl.pallas_call(
        _l0_kernel,
        grid=(2, T),
        in_specs=[
            pl.BlockSpec((1, B, DIM), emb_map),
            pl.BlockSpec((1, DIM, 4 * H), lambda di, t: (di, 0, 0)),
            pl.BlockSpec((1, H, 4 * H), lambda di, t: (di, 0, 0)),
            pl.BlockSpec((1, 1, 4 * H), lambda di, t: (di, 0, 0)),
        ],
        out_specs=pl.BlockSpec((1, 1, B, H), out_map),
        out_shape=jax.ShapeDtypeStruct((2, T, B, H), jnp.bfloat16),
        scratch_shapes=[
            pltpu.VMEM((B, H), jnp.bfloat16),
            pltpu.VMEM((B, H), jnp.float32),
        ],
    )(emb, wih2, whh2, b2)


def _l1_kernel(hf_ref, hb_ref, wa_ref, wb_ref, whh_ref, b_ref, out_ref,
               h_ref, c_ref):
    t = pl.program_id(0)

    @pl.when(t == 0)
    def _():
        h_ref[...] = jnp.zeros_like(h_ref)
        c_ref[...] = jnp.zeros_like(c_ref)

    gates = (
        jnp.dot(hf_ref[0], wa_ref[...], preferred_element_type=jnp.float32)
        + jnp.dot(hb_ref[0], wb_ref[...], preferred_element_type=jnp.float32)
        + jnp.dot(h_ref[...], whh_ref[...], preferred_element_type=jnp.float32)
        + b_ref[...]
    )
    h2, c2 = _lstm_gates(gates, c_ref[...])
    h2b = h2.astype(jnp.bfloat16)
    h_ref[...] = h2b
    c_ref[...] = c2

    @pl.when(t == T - 1)
    def _():
        out_ref[...] = h2b


def _layer1_fwd(hf0, hb0, wa, wb, whh, b):
    return pl.pallas_call(
        _l1_kernel,
        grid=(T,),
        in_specs=[
            pl.BlockSpec((1, B, H), lambda t: (t, 0, 0)),
            pl.BlockSpec((1, B, H), lambda t: (t, 0, 0)),
            pl.BlockSpec((H, 4 * H), lambda t: (0, 0)),
            pl.BlockSpec((H, 4 * H), lambda t: (0, 0)),
            pl.BlockSpec((H, 4 * H), lambda t: (0, 0)),
            pl.BlockSpec((1, 4 * H), lambda t: (0, 0)),
        ],
        out_specs=pl.BlockSpec((B, H), lambda t: (0, 0)),
        out_shape=jax.ShapeDtypeStruct((B, H), jnp.bfloat16),
        scratch_shapes=[
            pltpu.VMEM((B, H), jnp.bfloat16),
            pltpu.VMEM((B, H), jnp.float32),
        ],
    )(hf0, hb0, wa, wb, whh, b)


def _final_kernel(hfl_ref, hbl_ref, hf1_ref, wa_ref, wb_ref, b_ref,
                  wfa_ref, wfb_ref, bfc_ref, out_ref):
    gates = (
        jnp.dot(hfl_ref[...], wa_ref[...], preferred_element_type=jnp.float32)
        + jnp.dot(hbl_ref[...], wb_ref[...], preferred_element_type=jnp.float32)
        + b_ref[...]
    )
    h2, _ = _lstm_gates(gates, jnp.zeros((B, H), jnp.float32))
    h2 = h2.astype(jnp.bfloat16)
    out_ref[...] = (
        jnp.dot(hf1_ref[...], wfa_ref[...], preferred_element_type=jnp.float32)
        + jnp.dot(h2, wfb_ref[...], preferred_element_type=jnp.float32)
        + bfc_ref[...]
    )


def _final(hf0_last, hb0_last, hf1, wa, wb, b, wfa, wfb, bfc_row):
    npad = wfa.shape[1]
    return pl.pallas_call(
        _final_kernel,
        out_shape=jax.ShapeDtypeStruct((B, npad), jnp.float32),
    )(hf0_last, hb0_last, hf1, wa, wb, b, wfa, wfb, bfc_row)


def kernel(x, table, Wih0f, Whh0f, bih0f, bhh0f, Wih0b, Whh0b, bih0b, bhh0b,
           Wih1f, Whh1f, bih1f, bhh1f, Wih1b, Whh1b, bih1b, bhh1b, Wfc, bfc):
    idx = x.astype(jnp.int32).T.reshape(-1)
    emb = _sc_gather(table, idx).reshape(T, B, DIM)

    bf = jnp.bfloat16
    wih2 = jnp.stack([_halve_ifo(Wih0f.T), _halve_ifo(Wih0b.T)]).astype(bf)
    whh2 = jnp.stack([_halve_ifo(Whh0f.T), _halve_ifo(Whh0b.T)]).astype(bf)
    b2 = jnp.stack([_halve_ifo_bias((bih0f + bhh0f).reshape(1, -1)),
                    _halve_ifo_bias((bih0b + bhh0b).reshape(1, -1))])
    h0 = _layer0(emb, wih2, whh2, b2)
    hf0, hb0 = h0[0], h0[1]

    w1f = _halve_ifo(Wih1f.T).astype(bf)
    hf1 = _layer1_fwd(hf0, hb0, w1f[:H], w1f[H:],
                      _halve_ifo(Whh1f.T).astype(bf),
                      _halve_ifo_bias((bih1f + bhh1f).reshape(1, -1)))

    w1b = _halve_ifo(Wih1b.T).astype(bf)
    npad = 128
    wfc_t = jnp.zeros((2 * H, npad), jnp.float32).at[:, :CLASSES].set(Wfc.T)
    wfc_t = wfc_t.astype(bf)
    bfc_row = jnp.zeros((1, npad), jnp.float32).at[:, :CLASSES].set(bfc)
    logits = _final(hf0[T - 1], hb0[T - 1], hf1,
                    w1b[:H], w1b[H:],
                    _halve_ifo_bias((bih1b + bhh1b).reshape(1, -1)),
                    wfc_t[:H], wfc_t[H:], bfc_row)
    return logits[:, :CLASSES]
